# baseline (speedup 1.0000x reference)
import functools

import jax
import jax.numpy as jnp
import numpy as np
from jax import lax
from jax.experimental import pallas as pl
from jax.experimental.pallas import tpu as pltpu
from jax.experimental.pallas import tpu_sc as plsc

_MAX_SEQ = 4096
_BASE = 10000.0

_NC = 2
_NS = 16
_NW = _NC * _NS
_LANES = 16
_GATHER_CHUNK = 128


def _positional_encoding(max_len, d, base):
    pos = np.arange(max_len, dtype=np.float64)[:, None]
    i = np.arange(d, dtype=np.float64)[None, :]
    angle = pos / np.power(base, (2.0 * np.floor(i / 2.0)) / d)
    pe = np.zeros((max_len, d), dtype=np.float64)
    pe[:, 0::2] = np.sin(angle[:, 0::2])
    pe[:, 1::2] = np.cos(angle[:, 1::2])
    return pe.astype(np.float32)


@jax.jit
def _run(tokens, table):
    b, seq = tokens.shape
    d = table.shape[1]
    n_total = b * seq
    n_per = n_total // _NW
    n_ch = n_per // _GATHER_CHUNK
    idx_flat = tokens.reshape(n_total).astype(jnp.int32)
    pe = jnp.asarray(_positional_encoding(_MAX_SEQ, d, _BASE)[:seq])
    mesh = plsc.VectorSubcoreMesh(
        core_axis_name="c", subcore_axis_name="s",
        num_cores=_NC, num_subcores=_NS)

    @functools.partial(
        pl.kernel,
        out_type=jax.ShapeDtypeStruct((n_total, d), jnp.float32),
        mesh=mesh,
        scratch_types=[
            pltpu.VMEM((n_per,), jnp.int32),
            pltpu.VMEM((n_per, d), jnp.float32),
            pltpu.VMEM((n_per, d), jnp.float32),
            pltpu.SemaphoreType.DMA,
        ],
        compiler_params=pltpu.CompilerParams(use_tc_tiling_on_sc=False),
    )
    def emb_kernel(tok_hbm, table_hbm, pe_hbm, out_hbm, idx_v, rows_v, pe_v, sem):
        wid = lax.axis_index("s") * _NC + lax.axis_index("c")
        base = wid * n_per
        pos = lax.rem(base, seq)
        pltpu.sync_copy(tok_hbm.at[pl.ds(base, n_per)], idx_v)
        copies = []
        for j in range(n_ch):
            sl = pl.ds(j * _GATHER_CHUNK, _GATHER_CHUNK)
            copies.append(
                pltpu.async_copy(table_hbm.at[idx_v.at[sl]], rows_v.at[sl], sem))
        pltpu.sync_copy(pe_hbm.at[pl.ds(pos, n_per)], pe_v)
        for cp in copies:
            cp.wait()

        def add_body(r, carry):
            for j in range(d // _LANES):
                sl = pl.ds(j * _LANES, _LANES)
                rows_v[r, sl] = rows_v[r, sl] + pe_v[r, sl]
            return carry

        lax.fori_loop(0, n_per, add_body, 0)
        pltpu.sync_copy(rows_v, out_hbm.at[pl.ds(base, n_per)])

    return emb_kernel(idx_flat, table, pe).reshape(b, seq, d)


def kernel(tokens, embedding_table):
    return _run(tokens, embedding_table)


# speedup vs baseline: 1.6723x; 1.6723x over previous
import functools

import jax
import jax.numpy as jnp
import numpy as np
from jax import lax
from jax.experimental import pallas as pl
from jax.experimental.pallas import tpu as pltpu
from jax.experimental.pallas import tpu_sc as plsc

_MAX_SEQ = 4096
_BASE = 10000.0

_NC = 2
_NS = 16
_NW = _NC * _NS
_LANES = 16
_CH = 32


def _positional_encoding(max_len, d, base):
    pos = np.arange(max_len, dtype=np.float64)[:, None]
    i = np.arange(d, dtype=np.float64)[None, :]
    angle = pos / np.power(base, (2.0 * np.floor(i / 2.0)) / d)
    pe = np.zeros((max_len, d), dtype=np.float64)
    pe[:, 0::2] = np.sin(angle[:, 0::2])
    pe[:, 1::2] = np.cos(angle[:, 1::2])
    return pe.astype(np.float32)


@jax.jit
def _run(tokens, table):
    b, seq = tokens.shape
    d = table.shape[1]
    n_total = b * seq
    n_per = n_total // _NW
    n_ch = n_per // _CH
    idx_flat = tokens.reshape(n_total).astype(jnp.int32)
    pe = jnp.asarray(_positional_encoding(_MAX_SEQ, d, _BASE)[:seq])
    mesh = plsc.VectorSubcoreMesh(
        core_axis_name="c", subcore_axis_name="s",
        num_cores=_NC, num_subcores=_NS)

    @functools.partial(
        pl.kernel,
        out_type=jax.ShapeDtypeStruct((n_total, d), jnp.float32),
        mesh=mesh,
        scratch_types=[
            pltpu.VMEM((n_per,), jnp.int32),
            pltpu.VMEM((2, _CH, d), jnp.float32),
            pltpu.VMEM((n_per, d), jnp.float32),
            pltpu.SemaphoreType.DMA,
            pltpu.SemaphoreType.DMA,
            pltpu.SemaphoreType.DMA,
        ],
    )
    def emb_kernel(tok_hbm, table_hbm, pe_hbm, out_hbm,
                   idx_v, rows_v, pe_v, sem0, sem1, sem_pe):
        wid = lax.axis_index("s") * _NC + lax.axis_index("c")
        base = wid * n_per
        pos = lax.rem(base, seq)
        pltpu.sync_copy(tok_hbm.at[pl.ds(base, n_per)], idx_v)
        pe_cp = pltpu.async_copy(pe_hbm.at[pl.ds(pos, n_per)], pe_v, sem_pe)
        sems = (sem0, sem1)

        def fire(k):
            kb = k % 2
            cps = []
            for g in range(_CH // _LANES):
                tvec = idx_v[pl.ds(k * _CH + g * _LANES, _LANES)]
                for j in range(_LANES):
                    i = g * _LANES + j
                    cps.append(pltpu.async_copy(
                        table_hbm.at[tvec[j]], rows_v.at[kb, i], sems[kb]))
            return cps

        copies = [None, None]
        copies[0] = fire(0)
        pe_cp.wait()

        for k in range(n_ch):
            if k + 1 < n_ch:
                copies[(k + 1) % 2] = fire(k + 1)
            kb = k % 2
            for cp in copies[kb]:
                cp.wait()

            def add_body(i, carry, kb=kb, k=k):
                for c in range(d // _LANES):
                    sl = pl.ds(c * _LANES, _LANES)
                    rows_v[kb, i, sl] = (
                        rows_v[kb, i, sl] + pe_v[k * _CH + i, sl])
                return carry

            lax.fori_loop(0, _CH, add_body, 0)
            pltpu.sync_copy(rows_v.at[kb], out_hbm.at[pl.ds(base + k * _CH, _CH)])

    return emb_kernel(idx_flat, table, pe).reshape(b, seq, d)


def kernel(tokens, embedding_table):
    return _run(tokens, embedding_table)
